# Initial kernel scaffold; baseline (speedup 1.0000x reference)
import jax
import jax.numpy as jnp
from jax.experimental import pallas as pl


def kernel(p, x, ln1_g, ln1_b, Wqkv, Wproj, bproj, ln2_g, ln2_b, W1, b1, W2, b2, o):
    raise NotImplementedError("write your pallas kernel here")


# speedup vs baseline: 6.5028x; 6.5028x over previous
import jax
import jax.numpy as jnp
from jax.experimental import pallas as pl
from jax.experimental.pallas import tpu as pltpu

N = 4096
C = 256
K = 16
L = 4
BLK = 256
NB = N // BLK
SCALE = float(C) ** (-0.5)
NEG = -1e30


def _knn_mask_kernel(pb_ref, pt_ref, mask_ref):
    pb = pb_ref[...]
    pt = pt_ref[...]
    g = jax.lax.dot_general(pb, pt, (((1,), (0,)), ((), ())),
                            preferred_element_type=jnp.float32)
    sqr = jnp.sum(pb * pb, axis=1, keepdims=True)
    sqc = jnp.sum(pt * pt, axis=0, keepdims=True)
    d = sqr + sqc - 2.0 * g
    colid = jax.lax.broadcasted_iota(jnp.int32, (BLK, N), 1)
    m = jnp.zeros((BLK, N), jnp.float32)
    for _ in range(K):
        v = jnp.min(d, axis=1, keepdims=True)
        is_min = (d - v) <= 0.0
        col = jnp.min(jnp.where(is_min, colid, N), axis=1, keepdims=True)
        hit = (colid - col) == 0
        m = jnp.where(hit, 1.0, m)
        d = jnp.where(hit, jnp.float32(jnp.inf), d)
    mask_ref[...] = m.astype(jnp.int8)


def _ln(x, g, b):
    mu = jnp.mean(x, axis=1, keepdims=True)
    var = jnp.mean((x - mu) ** 2, axis=1, keepdims=True)
    return (x - mu) / jnp.sqrt(var + 1e-5) * g + b


def _qkv_kernel(x_ref, g_ref, b_ref, w_ref, q_ref, k_ref, v_ref):
    xn = _ln(x_ref[...], g_ref[...], b_ref[...])
    qkv = jnp.dot(xn, w_ref[...], preferred_element_type=jnp.float32)
    q_ref[...] = qkv[:, :C]
    k_ref[...] = qkv[:, C:2 * C]
    v_ref[...] = qkv[:, 2 * C:]


def _attn_mlp_kernel(q_ref, k_ref, v_ref, mask_ref, x_ref, wp_ref, bp_ref,
                     g2_ref, b2_ref, w1_ref, b1_ref, w2_ref, b2w_ref, o_ref):
    q = q_ref[...]
    s = jax.lax.dot_general(q, k_ref[...], (((1,), (1,)), ((), ())),
                            preferred_element_type=jnp.float32) * SCALE
    s = jnp.where(mask_ref[...].astype(jnp.float32) > 0.0, s, NEG)
    s = s - jnp.max(s, axis=1, keepdims=True)
    e = jnp.exp(s)
    a = e / jnp.sum(e, axis=1, keepdims=True)
    out = jnp.dot(a, v_ref[...], preferred_element_type=jnp.float32)
    out = jnp.dot(out, wp_ref[...], preferred_element_type=jnp.float32) + bp_ref[...]
    x1 = x_ref[...] + out
    xn2 = _ln(x1, g2_ref[...], b2_ref[...])
    h = jnp.dot(xn2, w1_ref[...], preferred_element_type=jnp.float32) + b1_ref[...]
    h = 0.5 * h * (1.0 + jax.lax.erf(h * (2.0 ** -0.5)))
    h = jnp.dot(h, w2_ref[...], preferred_element_type=jnp.float32) + b2w_ref[...]
    o_ref[...] = x1 + h


def _blk(shape):
    return pl.BlockSpec(shape, lambda i: (i,) + (0,) * (len(shape) - 1))


def _full(shape):
    return pl.BlockSpec(shape, lambda i: (0,) * len(shape))


def kernel(p, x, ln1_g, ln1_b, Wqkv, Wproj, bproj, ln2_g, ln2_b, W1, b1, W2,
           b2, o):
    pt = p.T
    mask = pl.pallas_call(
        _knn_mask_kernel,
        grid=(NB,),
        in_specs=[_blk((BLK, 3)), _full((3, N))],
        out_specs=_blk((BLK, N)),
        out_shape=jax.ShapeDtypeStruct((N, N), jnp.int8),
    )(p, pt)

    fC = jax.ShapeDtypeStruct((N, C), jnp.float32)
    for i in range(L):
        q, k, v = pl.pallas_call(
            _qkv_kernel,
            grid=(NB,),
            in_specs=[_blk((BLK, C)), _full((1, C)), _full((1, C)),
                      _full((C, 3 * C))],
            out_specs=[_blk((BLK, C))] * 3,
            out_shape=[fC, fC, fC],
        )(x, ln1_g[i][None], ln1_b[i][None], Wqkv[i])
        x = pl.pallas_call(
            _attn_mlp_kernel,
            grid=(NB,),
            in_specs=[_blk((BLK, C)), _full((N, C)), _full((N, C)),
                      _blk((BLK, N)), _blk((BLK, C)), _full((C, C)),
                      _full((1, C)), _full((1, C)), _full((1, C)),
                      _full((C, 4 * C)), _full((1, 4 * C)),
                      _full((4 * C, C)), _full((1, C))],
            out_specs=_blk((BLK, C)),
            out_shape=fC,
        )(q, k, v, mask, x, Wproj[i], bproj[i][None], ln2_g[i][None],
          ln2_b[i][None], W1[i], b1[i][None], W2[i], b2[i][None])
    return x
